# baseline (speedup 1.0000x reference)
import functools

import jax
import jax.numpy as jnp
from jax import lax
from jax.experimental import pallas as pl
from jax.experimental.pallas import tpu as pltpu
from jax.experimental.pallas import tpu_sc as plsc

_C, _T, _BR, _S, _SYN = 64, 16, 8, 16, 32
_NPREV = 8192
_ZSLOT = _NPREV
_TBL = _NPREV + 16
_NPAIR = _T * _BR

_info = plsc.get_sparse_core_info()
_NC = _info.num_cores
_NW = _NC * _info.num_subcores
_L = 16
_PPW = _NPAIR // _NW
_CG = _C // _L
_NBUF = 4

_EMASK = 0x0F0F0F0F
_SEGBIAS = 0x78787878
_BRBIAS = 0x7C7C7C7C
_ONES = 0x01010101


def _sc_body(tbl_hbm, idx_hbm, oe_hbm, oo_hbm,
             tbl_v, seg_v, oe_v, oo_v, sem_t, sem_a, sem_b):
    wid = lax.axis_index("s") * _NC + lax.axis_index("c")
    pair0 = wid * _PPW
    ct = pltpu.async_copy(tbl_hbm, tbl_v.at[pl.ds(0, _NPREV)], sem_t)
    sems = (sem_a, sem_b)

    _HS = _S // 2

    def blk_copy(blk, buf):
        flat = pair0 + blk // 2
        t = flat // _BR
        br = flat % _BR
        half = blk % 2
        return pltpu.async_copy(
            idx_hbm.at[t, br, pl.ds(half * _HS, _HS)], seg_v.at[buf],
            sems[buf])

    nblk = _PPW * 2
    cps = [blk_copy(0, 0), None]
    ct.wait()

    zero = jnp.zeros((_L,), jnp.int32)
    big = jnp.full((_L,), _ZSLOT, jnp.uint32)
    tbl_v[pl.ds(_NPREV, _L)] = zero

    accs = None
    for blk in range(nblk):
        buf = blk & 1
        if blk + 1 < nblk:
            cps[1 - buf] = blk_copy(blk + 1, 1 - buf)
        cps[buf].wait()
        if blk % 2 == 0:
            accs = [(zero, zero)] * _CG

        for cg in range(_CG):

            def seg_loop(s, carry, buf=buf, cg=cg):
                br_e, br_o = carry
                seg_e = zero
                seg_o = zero
                for k in range(4):
                    part = zero
                    for j in range(8):
                        raw = seg_v[buf, s, k * 8 + j, pl.ds(cg * _L, _L)]
                        safe = plsc.bitcast(
                            jnp.minimum(plsc.bitcast(raw, jnp.uint32), big),
                            jnp.int32)
                        part = part + plsc.load_gather(tbl_v, [safe])
                    seg_e = seg_e + (part & _EMASK)
                    seg_o = seg_o + ((part >> 4) & _EMASK)
                br_e = br_e + (((seg_e + _SEGBIAS) >> 7) & _ONES)
                br_o = br_o + (((seg_o + _SEGBIAS) >> 7) & _ONES)
                return br_e, br_o

            accs[cg] = lax.fori_loop(0, _HS, seg_loop, accs[cg])
        if blk % 2 == 1:
            pair = blk // 2
            for cg in range(_CG):
                br_e, br_o = accs[cg]
                off = (pair * _C) + cg * _L
                oe_v[pl.ds(off, _L)] = ((br_e + _BRBIAS) >> 7) & _ONES
                oo_v[pl.ds(off, _L)] = ((br_o + _BRBIAS) >> 7) & _ONES
    out0 = pair0 * _C
    pltpu.sync_copy(oe_v, oe_hbm.at[pl.ds(out0, _PPW * _C)])
    pltpu.sync_copy(oo_v, oo_hbm.at[pl.ds(out0, _PPW * _C)])


_sc_call = functools.partial(
    pl.kernel,
    mesh=plsc.VectorSubcoreMesh(core_axis_name="c", subcore_axis_name="s"),
    compiler_params=pltpu.CompilerParams(
        needs_layout_passes=False,
        disable_bounds_checks=True,
        disable_semaphore_checks=True,
        skip_device_barrier=True,
    ),
    out_type=[jax.ShapeDtypeStruct((_NPAIR * _C,), jnp.int32),
              jax.ShapeDtypeStruct((_NPAIR * _C,), jnp.int32)],
    scratch_types=[
        pltpu.VMEM((_TBL,), jnp.int32),
        pltpu.VMEM((2, _S // 2, _SYN, _C), jnp.int32),
        pltpu.VMEM((_PPW * _C,), jnp.int32),
        pltpu.VMEM((_PPW * _C,), jnp.int32),
        pltpu.SemaphoreType.DMA,
        pltpu.SemaphoreType.DMA,
        pltpu.SemaphoreType.DMA,
    ],
)(_sc_body)


def kernel(x, idx):
    prev = x != 0
    bits = prev.astype(jnp.int32)
    shifts = (jnp.arange(8, dtype=jnp.int32) * 4)[:, None]
    packed = jnp.sum(bits << shifts, axis=0)
    idx_t = jnp.transpose(idx, (1, 2, 3, 4, 0))
    oe, oo = _sc_call(packed, idx_t)
    packed_to = jnp.stack([oe, oo], axis=0)
    sh = (jnp.arange(4, dtype=jnp.int32) * 8)[:, None, None]
    bon = (packed_to[None] >> sh) & 1
    bon = bon.reshape(8, _T, _BR, _C)
    bon = jnp.transpose(bon, (0, 3, 1, 2)).astype(jnp.bool_)
    final = bon[:, :, 0].astype(jnp.int32)
    return (final, prev, bon)


# speedup vs baseline: 1.0017x; 1.0017x over previous
import functools

import jax
import jax.numpy as jnp
from jax import lax
from jax.experimental import pallas as pl
from jax.experimental.pallas import tpu as pltpu
from jax.experimental.pallas import tpu_sc as plsc

_C, _T, _BR, _S, _SYN = 64, 16, 8, 16, 32
_NPREV = 8192
_ZSLOT = _NPREV
_TBL = _NPREV + 16
_NPAIR = _T * _BR

_info = plsc.get_sparse_core_info()
_NC = _info.num_cores
_NW = _NC * _info.num_subcores
_L = 16
_PPW = _NPAIR // _NW
_CG = _C // _L
_NBUF = 4

_EMASK = 0x0F0F0F0F
_SEGBIAS = 0x78787878
_BRBIAS = 0x7C7C7C7C
_ONES = 0x01010101


def _sc_body(tbl_hbm, idx_hbm, oe_hbm, oo_hbm,
             tbl_v, seg_v, oe_v, oo_v, sem_t, sem_a, sem_b):
    wid = lax.axis_index("s") * _NC + lax.axis_index("c")
    pair0 = wid * _PPW
    ct = pltpu.async_copy(tbl_hbm, tbl_v.at[pl.ds(0, _NPREV)], sem_t)
    sems = (sem_a, sem_b)

    _HS = _S // 2

    def blk_copy(blk, buf):
        flat = pair0 + blk // 2
        t = flat // _BR
        br = flat % _BR
        half = blk % 2
        return pltpu.async_copy(
            idx_hbm.at[t, br, pl.ds(half * _HS, _HS)], seg_v.at[buf],
            sems[buf])

    nblk = _PPW * 2
    cps = [blk_copy(0, 0), None]
    ct.wait()

    zero = jnp.zeros((_L,), jnp.int32)
    big = jnp.full((_L,), _ZSLOT, jnp.uint32)
    tbl_v[pl.ds(_NPREV, _L)] = zero

    accs = None
    for blk in range(nblk):
        buf = blk & 1
        if blk + 1 < nblk:
            cps[1 - buf] = blk_copy(blk + 1, 1 - buf)
        cps[buf].wait()
        if blk % 2 == 0:
            accs = [(zero, zero)] * _CG

        for cg in range(_CG):

            def seg_loop(s, carry, buf=buf, cg=cg):
                br_e, br_o = carry
                seg_e = zero
                seg_o = zero
                for k in range(4):
                    part = zero
                    for j in range(8):
                        raw = seg_v[buf, s, k * 8 + j, pl.ds(cg * _L, _L)]
                        safe = plsc.bitcast(
                            jnp.minimum(plsc.bitcast(raw, jnp.uint32), big),
                            jnp.int32)
                        part = part + plsc.load_gather(tbl_v, [safe])
                    seg_e = seg_e + (part & _EMASK)
                    seg_o = seg_o + ((part >> 4) & _EMASK)
                br_e = br_e + (((seg_e + _SEGBIAS) >> 7) & _ONES)
                br_o = br_o + (((seg_o + _SEGBIAS) >> 7) & _ONES)
                return br_e, br_o

            accs[cg] = plsc.parallel_loop(
                0, _HS, carry=accs[cg])(seg_loop)
        if blk % 2 == 1:
            pair = blk // 2
            for cg in range(_CG):
                br_e, br_o = accs[cg]
                off = (pair * _C) + cg * _L
                oe_v[pl.ds(off, _L)] = ((br_e + _BRBIAS) >> 7) & _ONES
                oo_v[pl.ds(off, _L)] = ((br_o + _BRBIAS) >> 7) & _ONES
    out0 = pair0 * _C
    pltpu.sync_copy(oe_v, oe_hbm.at[pl.ds(out0, _PPW * _C)])
    pltpu.sync_copy(oo_v, oo_hbm.at[pl.ds(out0, _PPW * _C)])


_sc_call = functools.partial(
    pl.kernel,
    mesh=plsc.VectorSubcoreMesh(core_axis_name="c", subcore_axis_name="s"),
    compiler_params=pltpu.CompilerParams(
        needs_layout_passes=False,
        disable_bounds_checks=True,
        disable_semaphore_checks=True,
        skip_device_barrier=True,
    ),
    out_type=[jax.ShapeDtypeStruct((_NPAIR * _C,), jnp.int32),
              jax.ShapeDtypeStruct((_NPAIR * _C,), jnp.int32)],
    scratch_types=[
        pltpu.VMEM((_TBL,), jnp.int32),
        pltpu.VMEM((2, _S // 2, _SYN, _C), jnp.int32),
        pltpu.VMEM((_PPW * _C,), jnp.int32),
        pltpu.VMEM((_PPW * _C,), jnp.int32),
        pltpu.SemaphoreType.DMA,
        pltpu.SemaphoreType.DMA,
        pltpu.SemaphoreType.DMA,
    ],
)(_sc_body)


def kernel(x, idx):
    prev = x != 0
    bits = prev.astype(jnp.int32)
    shifts = (jnp.arange(8, dtype=jnp.int32) * 4)[:, None]
    packed = jnp.sum(bits << shifts, axis=0)
    idx_t = jnp.transpose(idx, (1, 2, 3, 4, 0))
    oe, oo = _sc_call(packed, idx_t)
    packed_to = jnp.stack([oe, oo], axis=0)
    sh = (jnp.arange(4, dtype=jnp.int32) * 8)[:, None, None]
    bon = (packed_to[None] >> sh) & 1
    bon = bon.reshape(8, _T, _BR, _C)
    bon = jnp.transpose(bon, (0, 3, 1, 2)).astype(jnp.bool_)
    final = bon[:, :, 0].astype(jnp.int32)
    return (final, prev, bon)


# speedup vs baseline: 1.1370x; 1.1350x over previous
import functools

import jax
import jax.numpy as jnp
from jax import lax
from jax.experimental import pallas as pl
from jax.experimental.pallas import tpu as pltpu
from jax.experimental.pallas import tpu_sc as plsc

_C, _T, _BR, _S, _SYN = 64, 16, 8, 16, 32
_NPREV = 8192
_ZSLOT = _NPREV
_TBL = _NPREV + 16
_NPAIR = _T * _BR

_info = plsc.get_sparse_core_info()
_NC = _info.num_cores
_NW = _NC * _info.num_subcores
_L = 16
_PPW = _NPAIR // _NW
_CG = _C // _L
_NBUF = 4

_EMASK = 0x0F0F0F0F
_SEGBIAS = 0x78787878
_BRBIAS = 0x7C7C7C7C
_ONES = 0x01010101


def _sc_body(tbl_hbm, idx_hbm, oe_hbm, oo_hbm,
             tbl_v, seg_v, acc_v, oe_v, oo_v, sem_t, sem_a, sem_b):
    wid = lax.axis_index("s") * _NC + lax.axis_index("c")
    pair0 = wid * _PPW
    ct = pltpu.async_copy(tbl_hbm, tbl_v.at[pl.ds(0, _NPREV)], sem_t)
    sems = (sem_a, sem_b)

    _HS = _S // 2

    def half_src(p, half):
        flat = pair0 + p
        t = flat // _BR
        br = flat % _BR
        return idx_hbm.at[t, br, pl.ds(half * _HS, _HS)]

    pltpu.async_copy(half_src(0, 0), seg_v.at[0], sem_a)
    pltpu.async_copy(half_src(0, 1), seg_v.at[1], sem_b)
    ct.wait()

    zero = jnp.zeros((_L,), jnp.int32)
    big = jnp.full((_L,), _ZSLOT, jnp.uint32)
    tbl_v[pl.ds(_NPREV, _L)] = zero

    def pair_body(p, carry):
        for half in (0, 1):
            sem = sems[half]
            pltpu.make_async_copy(
                half_src(p, half), seg_v.at[half], sem).wait()

            def cg_body(cg, _, half=half):
                if half == 0:
                    br_e = zero
                    br_o = zero
                else:
                    br_e = acc_v[0, pl.ds(cg * _L, _L)]
                    br_o = acc_v[1, pl.ds(cg * _L, _L)]

                def seg_loop(s, c, half=half, cg=cg):
                    br_e, br_o = c
                    seg_e = zero
                    seg_o = zero
                    for k in range(4):
                        part = zero
                        for j in range(8):
                            raw = seg_v[half, s, k * 8 + j,
                                        pl.ds(cg * _L, _L)]
                            safe = plsc.bitcast(
                                jnp.minimum(
                                    plsc.bitcast(raw, jnp.uint32), big),
                                jnp.int32)
                            part = part + plsc.load_gather(tbl_v, [safe])
                        seg_e = seg_e + (part & _EMASK)
                        seg_o = seg_o + ((part >> 4) & _EMASK)
                    br_e = br_e + (((seg_e + _SEGBIAS) >> 7) & _ONES)
                    br_o = br_o + (((seg_o + _SEGBIAS) >> 7) & _ONES)
                    return br_e, br_o

                br_e, br_o = plsc.parallel_loop(
                    0, _HS, unroll=2, carry=(br_e, br_o))(seg_loop)
                if half == 0:
                    acc_v[0, pl.ds(cg * _L, _L)] = br_e
                    acc_v[1, pl.ds(cg * _L, _L)] = br_o
                else:
                    off = p * _C + cg * _L
                    oe_v[pl.ds(off, _L)] = ((br_e + _BRBIAS) >> 7) & _ONES
                    oo_v[pl.ds(off, _L)] = ((br_o + _BRBIAS) >> 7) & _ONES
                return 0

            lax.fori_loop(0, _CG, cg_body, 0)

            @pl.when(p + 1 < _PPW)
            def _():
                pltpu.async_copy(half_src(p + 1, half), seg_v.at[half], sem)
        return carry

    lax.fori_loop(0, _PPW, pair_body, 0)
    out0 = pair0 * _C
    pltpu.sync_copy(oe_v, oe_hbm.at[pl.ds(out0, _PPW * _C)])
    pltpu.sync_copy(oo_v, oo_hbm.at[pl.ds(out0, _PPW * _C)])


_sc_call = functools.partial(
    pl.kernel,
    mesh=plsc.VectorSubcoreMesh(core_axis_name="c", subcore_axis_name="s"),
    compiler_params=pltpu.CompilerParams(
        needs_layout_passes=False,
        disable_bounds_checks=True,
        disable_semaphore_checks=True,
        skip_device_barrier=True,
    ),
    out_type=[jax.ShapeDtypeStruct((_NPAIR * _C,), jnp.int32),
              jax.ShapeDtypeStruct((_NPAIR * _C,), jnp.int32)],
    scratch_types=[
        pltpu.VMEM((_TBL,), jnp.int32),
        pltpu.VMEM((2, _S // 2, _SYN, _C), jnp.int32),
        pltpu.VMEM((2, _C), jnp.int32),
        pltpu.VMEM((_PPW * _C,), jnp.int32),
        pltpu.VMEM((_PPW * _C,), jnp.int32),
        pltpu.SemaphoreType.DMA,
        pltpu.SemaphoreType.DMA,
        pltpu.SemaphoreType.DMA,
    ],
)(_sc_body)


def kernel(x, idx):
    prev = x != 0
    bits = prev.astype(jnp.int32)
    shifts = (jnp.arange(8, dtype=jnp.int32) * 4)[:, None]
    packed = jnp.sum(bits << shifts, axis=0)
    idx_t = jnp.transpose(idx, (1, 2, 3, 4, 0))
    oe, oo = _sc_call(packed, idx_t)
    packed_to = jnp.stack([oe, oo], axis=0)
    sh = (jnp.arange(4, dtype=jnp.int32) * 8)[:, None, None]
    bon = (packed_to[None] >> sh) & 1
    bon = bon.reshape(8, _T, _BR, _C)
    bon = jnp.transpose(bon, (0, 3, 1, 2)).astype(jnp.bool_)
    final = bon[:, :, 0].astype(jnp.int32)
    return (final, prev, bon)


# speedup vs baseline: 1.1648x; 1.0245x over previous
import functools

import jax
import jax.numpy as jnp
from jax import lax
from jax.experimental import pallas as pl
from jax.experimental.pallas import tpu as pltpu
from jax.experimental.pallas import tpu_sc as plsc

_C, _T, _BR, _S, _SYN = 64, 16, 8, 16, 32
_NPREV = 8192
_ZSLOT = _NPREV
_TBL = _NPREV + 16
_NPAIR = _T * _BR

_info = plsc.get_sparse_core_info()
_NC = _info.num_cores
_NW = _NC * _info.num_subcores
_L = 16
_PPW = _NPAIR // _NW
_CG = _C // _L
_NBUF = 4

_EMASK = 0x0F0F0F0F
_SEGBIAS = 0x78787878
_BRBIAS = 0x7C7C7C7C
_ONES = 0x01010101


def _sc_body(tbl_hbm, idx_hbm, oe_hbm, oo_hbm,
             tbl_v, seg_v, acc_v, oe_v, oo_v, sem_t, sem_a, sem_b):
    wid = lax.axis_index("s") * _NC + lax.axis_index("c")
    pair0 = wid * _PPW
    ct = pltpu.async_copy(tbl_hbm, tbl_v.at[pl.ds(0, _NPREV)], sem_t)
    sems = (sem_a, sem_b)

    _HS = _S // 2

    def half_src(p, half):
        flat = pair0 + p
        t = flat // _BR
        br = flat % _BR
        return idx_hbm.at[t, br, pl.ds(half * _HS, _HS)]

    pltpu.async_copy(half_src(0, 0), seg_v.at[0], sem_a)
    pltpu.async_copy(half_src(0, 1), seg_v.at[1], sem_b)
    ct.wait()

    zero = jnp.zeros((_L,), jnp.int32)
    big = jnp.full((_L,), _ZSLOT, jnp.uint32)
    tbl_v[pl.ds(_NPREV, _L)] = zero

    def pair_body(p, carry):
        for half in (0, 1):
            sem = sems[half]
            pltpu.make_async_copy(
                half_src(p, half), seg_v.at[half], sem).wait()

            def cg_body(cg, _, half=half):
                if half == 0:
                    br_e = zero
                    br_o = zero
                else:
                    br_e = acc_v[0, pl.ds(cg * _L, _L)]
                    br_o = acc_v[1, pl.ds(cg * _L, _L)]

                def seg_loop(s, c, half=half, cg=cg):
                    br_e, br_o = c
                    seg_e = zero
                    seg_o = zero
                    for k in range(4):
                        part = zero
                        for j in range(8):
                            raw = seg_v[half, s, k * 8 + j,
                                        pl.ds(cg * _L, _L)]
                            safe = plsc.bitcast(
                                jnp.minimum(
                                    plsc.bitcast(raw, jnp.uint32), big),
                                jnp.int32)
                            part = part + plsc.load_gather(tbl_v, [safe])
                        seg_e = seg_e + (part & _EMASK)
                        seg_o = seg_o + ((part >> 4) & _EMASK)
                    br_e = br_e + (((seg_e + _SEGBIAS) >> 7) & _ONES)
                    br_o = br_o + (((seg_o + _SEGBIAS) >> 7) & _ONES)
                    return br_e, br_o

                br_e, br_o = plsc.parallel_loop(
                    0, _HS, carry=(br_e, br_o))(seg_loop)
                if half == 0:
                    acc_v[0, pl.ds(cg * _L, _L)] = br_e
                    acc_v[1, pl.ds(cg * _L, _L)] = br_o
                else:
                    off = p * _C + cg * _L
                    oe_v[pl.ds(off, _L)] = ((br_e + _BRBIAS) >> 7) & _ONES
                    oo_v[pl.ds(off, _L)] = ((br_o + _BRBIAS) >> 7) & _ONES
                return 0

            lax.fori_loop(0, _CG, cg_body, 0)

            @pl.when(p + 1 < _PPW)
            def _():
                pltpu.async_copy(half_src(p + 1, half), seg_v.at[half], sem)
        return carry

    lax.fori_loop(0, _PPW, pair_body, 0)
    out0 = pair0 * _C
    pltpu.sync_copy(oe_v, oe_hbm.at[pl.ds(out0, _PPW * _C)])
    pltpu.sync_copy(oo_v, oo_hbm.at[pl.ds(out0, _PPW * _C)])


_sc_call = functools.partial(
    pl.kernel,
    mesh=plsc.VectorSubcoreMesh(core_axis_name="c", subcore_axis_name="s"),
    compiler_params=pltpu.CompilerParams(
        needs_layout_passes=False,
        disable_bounds_checks=True,
        disable_semaphore_checks=True,
        skip_device_barrier=True,
    ),
    out_type=[jax.ShapeDtypeStruct((_NPAIR * _C,), jnp.int32),
              jax.ShapeDtypeStruct((_NPAIR * _C,), jnp.int32)],
    scratch_types=[
        pltpu.VMEM((_TBL,), jnp.int32),
        pltpu.VMEM((2, _S // 2, _SYN, _C), jnp.int32),
        pltpu.VMEM((2, _C), jnp.int32),
        pltpu.VMEM((_PPW * _C,), jnp.int32),
        pltpu.VMEM((_PPW * _C,), jnp.int32),
        pltpu.SemaphoreType.DMA,
        pltpu.SemaphoreType.DMA,
        pltpu.SemaphoreType.DMA,
    ],
)(_sc_body)


def kernel(x, idx):
    prev = x != 0
    bits = prev.astype(jnp.int32)
    shifts = (jnp.arange(8, dtype=jnp.int32) * 4)[:, None]
    packed = jnp.sum(bits << shifts, axis=0)
    idx_t = jnp.transpose(idx, (1, 2, 3, 4, 0))
    oe, oo = _sc_call(packed, idx_t)
    packed_to = jnp.stack([oe, oo], axis=0)
    sh = (jnp.arange(4, dtype=jnp.int32) * 8)[:, None, None]
    bon = (packed_to[None] >> sh) & 1
    bon = bon.reshape(8, _T, _BR, _C)
    bon = jnp.transpose(bon, (0, 3, 1, 2)).astype(jnp.bool_)
    final = bon[:, :, 0].astype(jnp.int32)
    return (final, prev, bon)


# speedup vs baseline: 1.1785x; 1.0118x over previous
import functools

import jax
import jax.numpy as jnp
from jax import lax
from jax.experimental import pallas as pl
from jax.experimental.pallas import tpu as pltpu
from jax.experimental.pallas import tpu_sc as plsc

_C, _T, _BR, _S, _SYN = 64, 16, 8, 16, 32
_NPREV = 8192
_ZSLOT = _NPREV
_TBL = _NPREV + 16
_NPAIR = _T * _BR

_info = plsc.get_sparse_core_info()
_NC = _info.num_cores
_NW = _NC * _info.num_subcores
_L = 16
_PPW = _NPAIR // _NW
_CG = _C // _L

_EMASK = 0x0F0F0F0F
_SEGBIAS = 0x78787878
_BRBIAS = 0x7C7C7C7C
_ONES = 0x01010101


def _sc_body(tbl_hbm, idx_hbm, oe_hbm, oo_hbm,
             tbl_v, seg_v, acc_v, oe_v, oo_v, sem_t, sem_a, sem_b):
    wid = lax.axis_index("s") * _NC + lax.axis_index("c")
    pair0 = wid * _PPW
    ct = pltpu.async_copy(tbl_hbm, tbl_v.at[pl.ds(0, _NPREV)], sem_t)
    sems = (sem_a, sem_b)

    _HS = _S // 2

    def half_src(p, half):
        flat = pair0 + p
        t = flat // _BR
        br = flat % _BR
        return idx_hbm.at[t, br, pl.ds(half * _HS, _HS)]

    pltpu.async_copy(half_src(0, 0), seg_v.at[0], sem_a)
    pltpu.async_copy(half_src(0, 1), seg_v.at[1], sem_b)
    ct.wait()

    zero = jnp.zeros((_L,), jnp.int32)
    big = jnp.full((_L,), _ZSLOT, jnp.uint32)
    tbl_v[pl.ds(_NPREV, _L)] = zero

    def pair_body(p, carry):
        for half in (0, 1):
            sem = sems[half]
            pltpu.make_async_copy(
                half_src(p, half), seg_v.at[half], sem).wait()

            def cg_body(cg, _, half=half):
                if half == 0:
                    br_e = zero
                    br_o = zero
                else:
                    br_e = acc_v[0, pl.ds(cg * _L, _L)]
                    br_o = acc_v[1, pl.ds(cg * _L, _L)]

                def seg_loop(s, c, half=half, cg=cg):
                    br_e, br_o = c
                    seg_e = zero
                    seg_o = zero
                    for k in range(4):
                        part = zero
                        for j in range(8):
                            raw = seg_v[half, s, k * 8 + j,
                                        pl.ds(cg * _L, _L)]
                            safe = plsc.bitcast(
                                jnp.minimum(
                                    plsc.bitcast(raw, jnp.uint32), big),
                                jnp.int32)
                            part = part + plsc.load_gather(tbl_v, [safe])
                        seg_e = seg_e + (part & _EMASK)
                        seg_o = seg_o + ((part >> 4) & _EMASK)
                    br_e = br_e + (((seg_e + _SEGBIAS) >> 7) & _ONES)
                    br_o = br_o + (((seg_o + _SEGBIAS) >> 7) & _ONES)
                    return br_e, br_o

                br_e, br_o = plsc.parallel_loop(
                    0, _HS, carry=(br_e, br_o))(seg_loop)
                if half == 0:
                    acc_v[0, pl.ds(cg * _L, _L)] = br_e
                    acc_v[1, pl.ds(cg * _L, _L)] = br_o
                else:
                    off = p * _C + cg * _L
                    oe_v[pl.ds(off, _L)] = ((br_e + _BRBIAS) >> 7) & _ONES
                    oo_v[pl.ds(off, _L)] = ((br_o + _BRBIAS) >> 7) & _ONES
                return 0

            lax.fori_loop(0, _CG, cg_body, 0)

            @pl.when(p + 1 < _PPW)
            def _():
                pltpu.async_copy(half_src(p + 1, half), seg_v.at[half], sem)
        return carry

    lax.fori_loop(0, _PPW, pair_body, 0)
    out0 = pair0 * _C
    pltpu.sync_copy(oe_v, oe_hbm.at[pl.ds(out0, _PPW * _C)])
    pltpu.sync_copy(oo_v, oo_hbm.at[pl.ds(out0, _PPW * _C)])


_sc_call = functools.partial(
    pl.kernel,
    mesh=plsc.VectorSubcoreMesh(core_axis_name="c", subcore_axis_name="s"),
    compiler_params=pltpu.CompilerParams(needs_layout_passes=False),
    out_type=[jax.ShapeDtypeStruct((_NPAIR * _C,), jnp.int32),
              jax.ShapeDtypeStruct((_NPAIR * _C,), jnp.int32)],
    scratch_types=[
        pltpu.VMEM((_TBL,), jnp.int32),
        pltpu.VMEM((2, _S // 2, _SYN, _C), jnp.int32),
        pltpu.VMEM((2, _C), jnp.int32),
        pltpu.VMEM((_PPW * _C,), jnp.int32),
        pltpu.VMEM((_PPW * _C,), jnp.int32),
        pltpu.SemaphoreType.DMA,
        pltpu.SemaphoreType.DMA,
        pltpu.SemaphoreType.DMA,
    ],
)(_sc_body)


def kernel(x, idx):
    prev = x != 0
    bits = prev.astype(jnp.int32)
    shifts = (jnp.arange(8, dtype=jnp.int32) * 4)[:, None]
    packed = jnp.sum(bits << shifts, axis=0)
    idx_t = jnp.transpose(idx, (1, 2, 3, 4, 0))
    oe, oo = _sc_call(packed, idx_t)
    packed_to = jnp.stack([oe, oo], axis=0)
    sh = (jnp.arange(4, dtype=jnp.int32) * 8)[:, None, None]
    bon = (packed_to[None] >> sh) & 1
    bon = bon.reshape(8, _T, _BR, _C)
    bon = jnp.transpose(bon, (0, 3, 1, 2)).astype(jnp.bool_)
    final = bon[:, :, 0].astype(jnp.int32)
    return (final, prev, bon)
